# Initial kernel scaffold; baseline (speedup 1.0000x reference)
import jax
import jax.numpy as jnp
from jax.experimental import pallas as pl


def kernel(h_src, h_dst, edge_index):
    raise NotImplementedError("write your pallas kernel here")


# speedup vs baseline: 4.5626x; 4.5626x over previous
import functools

import jax
import jax.numpy as jnp
from jax import lax
from jax.experimental import pallas as pl
from jax.experimental.pallas import tpu as pltpu
from jax.experimental.pallas import tpu_sc as plsc

N = 50000
D = 64
E = 800000

NQ = 4
FQ = 16
NT = 16
NC = 2

CHUNK = 128
CPB = 8
BLOCKS = 50
EPT = CHUNK * CPB * BLOCKS
EP = EPT * NT
PAD = EP - E

NP = 50048
ACC_ROWS = 2 * NP
DUMMY = 2 * N
ZROWS = 368
NPT = NP // NT
WCHUNK = 184


def _sc_body(hq, ig_src, ig_dst, is_dst2, is_src2, out_hbm, rst_hbm,
             acc, ib_g, ib_s, ib_sp, rows, ones, zb, inb, outb, sem0, sem1):
  cid = lax.axis_index("c")
  sid = lax.axis_index("s")
  sems = (sem0, sem1)

  def _init_ones(i, _):
    ones[i, :] = jnp.ones((FQ,), jnp.float32)
    return 0
  lax.fori_loop(0, CHUNK, _init_ones, 0)

  def _init_z(i, _):
    zb[i, :] = jnp.zeros((FQ,), jnp.float32)
    return 0
  lax.fori_loop(0, ZROWS, _init_z, 0)

  acc_base = sid * (ACC_ROWS // NT)
  for z in range(ACC_ROWS // NT // ZROWS):
    pltpu.sync_copy(zb, acc.at[pl.ds(acc_base + z * ZROWS, ZROWS)])
  plsc.subcore_barrier()

  def stage(g_tbl, ig, is2, qb):
    def block(b, _):
      row0 = sid * (BLOCKS * CPB) + b * CPB
      pltpu.sync_copy(ig.at[pl.ds(row0, CPB)], ib_g)
      pltpu.sync_copy(is2.at[pl.ds(row0, CPB)], ib_s)
      for r in range(CPB):
        for k in range(CHUNK // 16):
          sl = (r, pl.ds(k * 16, 16))
          ib_g[sl] = ib_g[sl] + qb
          ib_sp[sl] = ib_s[sl] + 1
      cps = [None, None]
      cps[0] = pltpu.async_copy(g_tbl.at[ib_g.at[0]], rows.at[0], sems[0])
      for j in range(CPB):
        if j + 1 < CPB:
          nb = (j + 1) % 2
          cps[nb] = pltpu.async_copy(
              g_tbl.at[ib_g.at[j + 1]], rows.at[nb], sems[nb])
        cps[j % 2].wait()
        pltpu.sync_copy(rows.at[j % 2], acc.at[ib_s.at[j]], add=True)
        pltpu.sync_copy(ones, acc.at[ib_sp.at[j]], add=True)
      return 0
    lax.fori_loop(0, BLOCKS, block, 0)

  def writeout(dst_hbm, qb):
    for w in range(NPT // WCHUNK):
      node0 = sid * NPT + w * WCHUNK
      pltpu.sync_copy(acc.at[pl.ds(2 * node0, 2 * WCHUNK)], inb)
      def norm(i, _):
        feat = inb[2 * i, :]
        deg = inb[2 * i + 1, :]
        outb[i, :] = feat / jnp.maximum(deg, 1.0)
        return 0
      lax.fori_loop(0, WCHUNK, norm, 0)
      pltpu.sync_copy(outb, dst_hbm.at[pl.ds(qb + node0, WCHUNK)])
      pltpu.sync_copy(zb.at[pl.ds(0, 2 * WCHUNK)],
                      acc.at[pl.ds(2 * node0, 2 * WCHUNK)])

  def one_pass(p, _):
    qb = p * (2 * NP) + cid * NP
    stage(hq, ig_src, is_dst2, qb)
    plsc.subcore_barrier()
    writeout(rst_hbm, qb)
    plsc.subcore_barrier()
    stage(rst_hbm, ig_dst, is_src2, qb)
    plsc.subcore_barrier()
    writeout(out_hbm, qb)
    plsc.subcore_barrier()
    return 0

  lax.fori_loop(0, NC, one_pass, 0)


@functools.partial(jax.jit, static_argnames=())
def _run(hq, g_src, g_dst, s_dst2, s_src2):
  mesh = plsc.VectorSubcoreMesh(core_axis_name="c", subcore_axis_name="s")
  f = pl.kernel(
      _sc_body,
      out_type=[
          jax.ShapeDtypeStruct((NQ * NP, FQ), jnp.float32),
          jax.ShapeDtypeStruct((NQ * NP, FQ), jnp.float32),
      ],
      mesh=mesh,
      scratch_types=[
          pltpu.VMEM_SHARED((ACC_ROWS, FQ), jnp.float32),
          pltpu.VMEM((CPB, CHUNK), jnp.int32),
          pltpu.VMEM((CPB, CHUNK), jnp.int32),
          pltpu.VMEM((CPB, CHUNK), jnp.int32),
          pltpu.VMEM((2, CHUNK, FQ), jnp.float32),
          pltpu.VMEM((CHUNK, FQ), jnp.float32),
          pltpu.VMEM((ZROWS, FQ), jnp.float32),
          pltpu.VMEM((2 * WCHUNK, FQ), jnp.float32),
          pltpu.VMEM((WCHUNK, FQ), jnp.float32),
          pltpu.SemaphoreType.DMA,
          pltpu.SemaphoreType.DMA,
      ],
      compiler_params=pltpu.CompilerParams(use_tc_tiling_on_sc=False),
  )
  return f(hq, g_src, g_dst, s_dst2, s_src2)


def kernel(h_src, h_dst, edge_index):
  del h_dst
  src = edge_index[0].astype(jnp.int32)
  dst = edge_index[1].astype(jnp.int32)
  pad_i = jnp.arange(PAD, dtype=jnp.int32)
  padg = pad_i % 16
  pads = DUMMY + 2 * (pad_i % 8)
  g_src = jnp.concatenate([src, padg]).reshape(EP // CHUNK, CHUNK)
  g_dst = jnp.concatenate([dst, padg]).reshape(EP // CHUNK, CHUNK)
  s_dst2 = jnp.concatenate([2 * dst, pads]).reshape(EP // CHUNK, CHUNK)
  s_src2 = jnp.concatenate([2 * src, pads]).reshape(EP // CHUNK, CHUNK)
  hq = h_src.reshape(N, NQ, FQ).transpose(1, 0, 2)
  hq = jnp.pad(hq, ((0, 0), (0, NP - N), (0, 0))).reshape(NQ * NP, FQ)
  out_q, _ = _run(hq, g_src, g_dst, s_dst2, s_src2)
  return out_q.reshape(NQ, NP, FQ)[:, :N].transpose(1, 0, 2).reshape(N, D)


# speedup vs baseline: 5.8776x; 1.2882x over previous
import functools

import jax
import jax.numpy as jnp
from jax import lax
from jax.experimental import pallas as pl
from jax.experimental.pallas import tpu as pltpu
from jax.experimental.pallas import tpu_sc as plsc

N = 50000
D = 64
E = 800000

NQ = 4
FQ = 16
NT = 16
NC = 2

CHUNK = 128
CPB = 8
BLOCKS = 50
RB = 4
EPT = CHUNK * CPB * BLOCKS
EP = EPT * NT
PAD = EP - E
ROWS_PER_Q = EP // CHUNK

NP = 50048
ACC_ROWS = 2 * NP
DUMMY = 2 * N
ZROWS = 368
NPT = NP // NT
WCHUNK = 92
NW = NPT // WCHUNK


def _sc_body(hq, ig_src, ig_dst, is_dst2, is_src2, out_hbm, rst_hbm,
             acc, ib_g, ib_s, ib_sp, rows, ones, zb, inb, outb, sems):
  cid = lax.axis_index("c")
  sid = lax.axis_index("s")
  sem_g, sem_f, sem_o, sem_i, sem_w, sem_z = sems

  def _init_ones(i, _):
    ones[i, :] = jnp.ones((FQ,), jnp.float32)
    return 0
  lax.fori_loop(0, CHUNK, _init_ones, 0)

  def _init_z(i, _):
    zb[i, :] = jnp.zeros((FQ,), jnp.float32)
    return 0
  lax.fori_loop(0, ZROWS, _init_z, 0)

  acc_base = sid * (ACC_ROWS // NT)
  for z in range(ACC_ROWS // NT // ZROWS):
    pltpu.sync_copy(zb, acc.at[pl.ds(acc_base + z * ZROWS, ZROWS)])
  plsc.subcore_barrier()

  def stage(g_tbl, ig, is2, qrow):
    def block(b, _):
      grow0 = qrow + sid * (BLOCKS * CPB) + b * CPB
      srow0 = sid * (BLOCKS * CPB) + b * CPB
      ci0 = pltpu.async_copy(ig.at[pl.ds(grow0, CPB)], ib_g, sem_i[0])
      ci1 = pltpu.async_copy(is2.at[pl.ds(srow0, CPB)], ib_s, sem_i[1])
      ci0.wait()
      ci1.wait()
      for r in range(CPB):
        for k in range(CHUNK // 16):
          sl = (r, pl.ds(k * 16, 16))
          ib_sp[sl] = ib_s[sl] + 1
      gcp = [None] * RB
      scf = [None] * RB
      sco = [None, None]
      for j in range(RB - 1):
        gcp[j] = pltpu.async_copy(g_tbl.at[ib_g.at[j]], rows.at[j], sem_g[j])
      for j in range(CPB):
        jn = j + RB - 1
        if jn < CPB:
          bn = jn % RB
          if scf[bn] is not None:
            scf[bn].wait()
          gcp[bn] = pltpu.async_copy(
              g_tbl.at[ib_g.at[jn]], rows.at[bn], sem_g[bn])
        gcp[j % RB].wait()
        if sco[j % 2] is not None:
          sco[j % 2].wait()
        scf[j % RB] = pltpu.async_copy(
            rows.at[j % RB], acc.at[ib_s.at[j]], sem_f[j % RB], add=True)
        sco[j % 2] = pltpu.async_copy(
            ones, acc.at[ib_sp.at[j]], sem_o[j % 2], add=True)
      for x in scf:
        if x is not None:
          x.wait()
      for x in sco:
        if x is not None:
          x.wait()
      return 0
    lax.fori_loop(0, BLOCKS, block, 0)

  def writeout(dst_hbm, qb):
    icp = [None, None]
    ocp = [None, None]
    zcp = [None, None]
    node00 = sid * NPT
    icp[0] = pltpu.async_copy(
        acc.at[pl.ds(2 * node00, 2 * WCHUNK)], inb.at[0], sem_w[0])
    for w in range(NW):
      pb = w % 2
      nb = (w + 1) % 2
      node0 = node00 + w * WCHUNK
      if w + 1 < NW:
        icp[nb] = pltpu.async_copy(
            acc.at[pl.ds(2 * (node0 + WCHUNK), 2 * WCHUNK)], inb.at[nb],
            sem_w[nb])
      icp[pb].wait()
      if ocp[pb] is not None:
        ocp[pb].wait()
      def norm(i, _):
        feat = inb[pb, 2 * i, :]
        deg = inb[pb, 2 * i + 1, :]
        outb[pb, i, :] = feat / jnp.maximum(deg, 1.0)
        return 0
      lax.fori_loop(0, WCHUNK, norm, 0)
      ocp[pb] = pltpu.async_copy(
          outb.at[pb], dst_hbm.at[pl.ds(qb + node0, WCHUNK)], sem_w[2 + pb])
      if zcp[pb] is not None:
        zcp[pb].wait()
      zcp[pb] = pltpu.async_copy(
          zb.at[pl.ds(0, 2 * WCHUNK)],
          acc.at[pl.ds(2 * node0, 2 * WCHUNK)], sem_z[pb])
    for x in ocp:
      if x is not None:
        x.wait()
    for x in zcp:
      if x is not None:
        x.wait()

  def one_pass(p, _):
    q = 2 * p + cid
    qb = q * NP
    qrow = q * ROWS_PER_Q
    stage(hq, ig_src, is_dst2, qrow)
    plsc.subcore_barrier()
    writeout(rst_hbm, qb)
    plsc.subcore_barrier()
    stage(rst_hbm, ig_dst, is_src2, qrow)
    plsc.subcore_barrier()
    writeout(out_hbm, qb)
    plsc.subcore_barrier()
    return 0

  lax.fori_loop(0, NC, one_pass, 0)


@functools.partial(jax.jit, static_argnames=())
def _run(hq, g_src, g_dst, s_dst2, s_src2):
  mesh = plsc.VectorSubcoreMesh(core_axis_name="c", subcore_axis_name="s")
  f = pl.kernel(
      _sc_body,
      out_type=[
          jax.ShapeDtypeStruct((NQ * NP, FQ), jnp.float32),
          jax.ShapeDtypeStruct((NQ * NP, FQ), jnp.float32),
      ],
      mesh=mesh,
      scratch_types=[
          pltpu.VMEM_SHARED((ACC_ROWS, FQ), jnp.float32),
          pltpu.VMEM((CPB, CHUNK), jnp.int32),
          pltpu.VMEM((CPB, CHUNK), jnp.int32),
          pltpu.VMEM((CPB, CHUNK), jnp.int32),
          pltpu.VMEM((RB, CHUNK, FQ), jnp.float32),
          pltpu.VMEM((CHUNK, FQ), jnp.float32),
          pltpu.VMEM((ZROWS, FQ), jnp.float32),
          pltpu.VMEM((2, 2 * WCHUNK, FQ), jnp.float32),
          pltpu.VMEM((2, WCHUNK, FQ), jnp.float32),
          (
              [pltpu.SemaphoreType.DMA] * RB,
              [pltpu.SemaphoreType.DMA] * RB,
              [pltpu.SemaphoreType.DMA] * 2,
              [pltpu.SemaphoreType.DMA] * 2,
              [pltpu.SemaphoreType.DMA] * 4,
              [pltpu.SemaphoreType.DMA] * 2,
          ),
      ],
      compiler_params=pltpu.CompilerParams(use_tc_tiling_on_sc=False),
  )
  return f(hq, g_src, g_dst, s_dst2, s_src2)


def kernel(h_src, h_dst, edge_index):
  del h_dst
  src = edge_index[0].astype(jnp.int32)
  dst = edge_index[1].astype(jnp.int32)
  pad_i = jnp.arange(PAD, dtype=jnp.int32)
  padg = pad_i % 16
  pads = DUMMY + 2 * (pad_i % 8)
  qoff = (jnp.arange(NQ, dtype=jnp.int32) * NP)[:, None]
  g_src = (jnp.concatenate([src, padg])[None, :] + qoff).reshape(
      NQ * ROWS_PER_Q, CHUNK)
  g_dst = (jnp.concatenate([dst, padg])[None, :] + qoff).reshape(
      NQ * ROWS_PER_Q, CHUNK)
  s_dst2 = jnp.concatenate([2 * dst, pads]).reshape(ROWS_PER_Q, CHUNK)
  s_src2 = jnp.concatenate([2 * src, pads]).reshape(ROWS_PER_Q, CHUNK)
  hq = h_src.reshape(N, NQ, FQ).transpose(1, 0, 2)
  hq = jnp.pad(hq, ((0, 0), (0, NP - N), (0, 0))).reshape(NQ * NP, FQ)
  out_q, _ = _run(hq, g_src, g_dst, s_dst2, s_src2)
  return out_q.reshape(NQ, NP, FQ)[:, :N].transpose(1, 0, 2).reshape(N, D)
